# baseline (speedup 1.0000x reference)
import jax
import jax.numpy as jnp
from jax.experimental import pallas as pl
from jax.experimental.pallas import tpu as pltpu

_ROWS = 96
_COLS = 16384
_BLOCK_ROWS = 4


def _add1_block(x_ref, o_ref):
    o_ref[...] = x_ref[...] + 1.0


def _outer(x_hbm, o_hbm):
    pltpu.emit_pipeline(
        _add1_block,
        grid=(_ROWS // _BLOCK_ROWS,),
        in_specs=[
            pl.BlockSpec(
                (_BLOCK_ROWS, _COLS),
                lambda i: (i, 0),
                pipeline_mode=pl.Buffered(buffer_count=2),
            )
        ],
        out_specs=[
            pl.BlockSpec(
                (_BLOCK_ROWS, _COLS),
                lambda i: (i, 0),
                pipeline_mode=pl.Buffered(buffer_count=2),
            )
        ],
    )(x_hbm, o_hbm)


def kernel(input_xyzs):
    b, n, c = input_xyzs.shape
    x = jnp.transpose(input_xyzs, (2, 0, 1)).reshape(c * b, n)
    out = pl.pallas_call(
        _outer,
        out_shape=jax.ShapeDtypeStruct((c * b, n), jnp.float32),
        in_specs=[pl.BlockSpec(memory_space=pl.ANY)],
        out_specs=pl.BlockSpec(memory_space=pl.ANY),
    )(x)
    return jnp.transpose(out.reshape(c, b, n), (1, 2, 0))


# speedup vs baseline: 2.2564x; 2.2564x over previous
import jax
import jax.numpy as jnp
from jax.experimental import pallas as pl
from jax.experimental.pallas import tpu as pltpu

_ROWS = 96
_COLS = 16384
_BLOCK_ROWS = 16


def _add1_block(x_ref, o_ref):
    o_ref[...] = x_ref[...] + 1.0


def _outer(x_hbm, o_hbm):
    pltpu.emit_pipeline(
        _add1_block,
        grid=(_ROWS // _BLOCK_ROWS,),
        in_specs=[
            pl.BlockSpec(
                (_BLOCK_ROWS, _COLS),
                lambda i: (i, 0),
                pipeline_mode=pl.Buffered(buffer_count=2),
            )
        ],
        out_specs=[
            pl.BlockSpec(
                (_BLOCK_ROWS, _COLS),
                lambda i: (i, 0),
                pipeline_mode=pl.Buffered(buffer_count=2),
            )
        ],
    )(x_hbm, o_hbm)


def kernel(input_xyzs):
    b, n, c = input_xyzs.shape
    x = jnp.transpose(input_xyzs, (2, 0, 1)).reshape(c * b, n)
    out = pl.pallas_call(
        _outer,
        out_shape=jax.ShapeDtypeStruct((c * b, n), jnp.float32),
        in_specs=[pl.BlockSpec(memory_space=pl.ANY)],
        out_specs=pl.BlockSpec(memory_space=pl.ANY),
    )(x)
    return jnp.transpose(out.reshape(c, b, n), (1, 2, 0))


# speedup vs baseline: 2.5139x; 1.1141x over previous
import jax
import jax.numpy as jnp
from jax.experimental import pallas as pl
from jax.experimental.pallas import tpu as pltpu

_ROWS = 96
_COLS = 16384
_BLOCK_ROWS = 32


def _add1_block(x_ref, o_ref):
    o_ref[...] = x_ref[...] + 1.0


def _outer(x_hbm, o_hbm):
    pltpu.emit_pipeline(
        _add1_block,
        grid=(_ROWS // _BLOCK_ROWS,),
        in_specs=[
            pl.BlockSpec(
                (_BLOCK_ROWS, _COLS),
                lambda i: (i, 0),
                pipeline_mode=pl.Buffered(buffer_count=2),
            )
        ],
        out_specs=[
            pl.BlockSpec(
                (_BLOCK_ROWS, _COLS),
                lambda i: (i, 0),
                pipeline_mode=pl.Buffered(buffer_count=2),
            )
        ],
    )(x_hbm, o_hbm)


def kernel(input_xyzs):
    b, n, c = input_xyzs.shape
    x = jnp.transpose(input_xyzs, (2, 0, 1)).reshape(c * b, n)
    out = pl.pallas_call(
        _outer,
        out_shape=jax.ShapeDtypeStruct((c * b, n), jnp.float32),
        in_specs=[pl.BlockSpec(memory_space=pl.ANY)],
        out_specs=pl.BlockSpec(memory_space=pl.ANY),
    )(x)
    return jnp.transpose(out.reshape(c, b, n), (1, 2, 0))


# speedup vs baseline: 3.2627x; 1.2978x over previous
import jax
import jax.numpy as jnp
from jax.experimental import pallas as pl
from jax.experimental.pallas import tpu as pltpu

_ROWS = 96
_COLS = 16384
_BLOCK_ROWS = 48


def _add1_block(x_ref, o_ref):
    o_ref[...] = x_ref[...] + 1.0


def _outer(x_hbm, o_hbm):
    pltpu.emit_pipeline(
        _add1_block,
        grid=(_ROWS // _BLOCK_ROWS,),
        in_specs=[
            pl.BlockSpec(
                (_BLOCK_ROWS, _COLS),
                lambda i: (i, 0),
                pipeline_mode=pl.Buffered(buffer_count=2),
            )
        ],
        out_specs=[
            pl.BlockSpec(
                (_BLOCK_ROWS, _COLS),
                lambda i: (i, 0),
                pipeline_mode=pl.Buffered(buffer_count=2),
            )
        ],
    )(x_hbm, o_hbm)


def kernel(input_xyzs):
    b, n, c = input_xyzs.shape
    x = jnp.transpose(input_xyzs, (2, 0, 1)).reshape(c * b, n)
    out = pl.pallas_call(
        _outer,
        out_shape=jax.ShapeDtypeStruct((c * b, n), jnp.float32),
        in_specs=[pl.BlockSpec(memory_space=pl.ANY)],
        out_specs=pl.BlockSpec(memory_space=pl.ANY),
    )(x)
    return jnp.transpose(out.reshape(c, b, n), (1, 2, 0))
